# Initial kernel scaffold; baseline (speedup 1.0000x reference)
import jax
import jax.numpy as jnp
from jax.experimental import pallas as pl


def kernel(x, edge_index, weight, W, b):
    raise NotImplementedError("write your pallas kernel here")


# speedup vs baseline: 8.5863x; 8.5863x over previous
import functools

import jax
import jax.numpy as jnp
from jax import lax
from jax.experimental import pallas as pl
from jax.experimental.pallas import tpu as pltpu, tpu_sc as plsc

N = 10000
E = 320000
D = 128

NC, NS, L = 2, 16, 16
NW = NC * NS
CH = 128
CPW = 80
EPAD = NW * CPW * CH
NCHUNK = NW * CPW
CPT_DEG = NCHUNK // NS
NPAD = 10240
NPT = NPAD // NS
G = 4
NG_DEG = CPT_DEG // G
NG = CPW // G


def _rsqrt_vec(d):
    i = plsc.bitcast(d, jnp.int32)
    i = jnp.full((L,), 0x5F3759DF, dtype=jnp.int32) - lax.shift_right_logical(i, 1)
    y = plsc.bitcast(i, jnp.float32)
    half_d = d * 0.5
    for _ in range(3):
        y = y * (1.5 - half_d * y * y)
    return jnp.where(d > 0.0, y, 0.0)


def _sc_aggregate(x, colp, wp, rowp):
    mesh = plsc.VectorSubcoreMesh(core_axis_name="c", subcore_axis_name="s")

    @functools.partial(
        pl.kernel,
        mesh=mesh,
        out_type=jax.ShapeDtypeStruct((NC, NPAD, D), jnp.float32),
        scratch_types=[
            pltpu.VMEM((NPAD,), jnp.float32),
            pltpu.VMEM((CH,), jnp.float32),
            pltpu.VMEM((2, G, CH), jnp.int32),
            pltpu.VMEM((2, G, CH), jnp.int32),
            pltpu.VMEM((2, G, CH), jnp.float32),
            pltpu.VMEM((2, CH, D), jnp.float32),
            pltpu.VMEM_SHARED((NPAD, D), jnp.float32),
            pltpu.VMEM_SHARED((NPAD,), jnp.float32),
            pltpu.VMEM_SHARED((NPAD,), jnp.float32),
            pltpu.SemaphoreType.DMA,
            pltpu.SemaphoreType.DMA,
            pltpu.SemaphoreType.DMA,
        ],
        compiler_params=pltpu.CompilerParams(needs_layout_passes=False),
    )
    def body(x_hbm, col_hbm, w_hbm, row_hbm, out_hbm,
             dinv_v, wch_v, rowg_v, colg_v, weg_v, xbuf_v,
             agg_sh, deg_sh, dinv_sh, semg0, semg1, seme):
        cid = lax.axis_index("c")
        sid = lax.axis_index("s")
        wid = sid * NC + cid

        def zero_xbuf(r, _):
            for g in range(D // L):
                xbuf_v[0, r, pl.ds(g * L, L)] = jnp.zeros((L,), jnp.float32)
            return 0
        lax.fori_loop(0, CH, zero_xbuf, 0)
        for k in range(NPT // CH):
            pltpu.sync_copy(xbuf_v.at[0],
                            agg_sh.at[pl.ds(sid * NPT + k * CH, CH)])
        for g in range(CH // L):
            wch_v[pl.ds(g * L, L)] = jnp.zeros((L,), jnp.float32)
        for k in range(NPT // CH):
            pltpu.sync_copy(wch_v, deg_sh.at[pl.ds(sid * NPT + k * CH, CH)])
        plsc.subcore_barrier()

        def start_deg_group(g, buf):
            base = sid * CPT_DEG + g * G
            pltpu.make_async_copy(
                col_hbm.at[pl.ds(base, G)], colg_v.at[buf], seme).start()
            pltpu.make_async_copy(
                w_hbm.at[pl.ds(base, G)], weg_v.at[buf], seme).start()

        def wait_deg_group(g, buf):
            base = sid * CPT_DEG + g * G
            pltpu.make_async_copy(
                col_hbm.at[pl.ds(base, G)], colg_v.at[buf], seme).wait()
            pltpu.make_async_copy(
                w_hbm.at[pl.ds(base, G)], weg_v.at[buf], seme).wait()

        def deg_group(g, buf):
            wait_deg_group(g, buf)
            for jj in range(G):
                pltpu.sync_copy(weg_v.at[buf, jj],
                                deg_sh.at[colg_v.at[buf, jj]], add=True)

            @pl.when(g + 2 < NG_DEG)
            def _():
                start_deg_group(g + 2, buf)

        start_deg_group(0, 0)
        start_deg_group(1, 1)

        def deg_pair(i, _):
            deg_group(i * 2, 0)
            deg_group(i * 2 + 1, 1)
            return 0
        lax.fori_loop(0, NG_DEG // 2, deg_pair, 0)
        plsc.subcore_barrier()

        def dinv_piece(k, _):
            pltpu.sync_copy(deg_sh.at[pl.ds(sid * NPT + k * CH, CH)], wch_v)
            for g in range(CH // L):
                wch_v[pl.ds(g * L, L)] = _rsqrt_vec(wch_v[pl.ds(g * L, L)])
            pltpu.sync_copy(wch_v, dinv_sh.at[pl.ds(sid * NPT + k * CH, CH)])
            return 0
        lax.fori_loop(0, NPT // CH, dinv_piece, 0)
        plsc.subcore_barrier()
        pltpu.sync_copy(dinv_sh, dinv_v)

        def start_edge_group(g, buf):
            base = wid * CPW + g * G
            pltpu.make_async_copy(
                row_hbm.at[pl.ds(base, G)], rowg_v.at[buf], seme).start()
            pltpu.make_async_copy(
                col_hbm.at[pl.ds(base, G)], colg_v.at[buf], seme).start()
            pltpu.make_async_copy(
                w_hbm.at[pl.ds(base, G)], weg_v.at[buf], seme).start()

        def wait_edge_group(g, buf):
            base = wid * CPW + g * G
            pltpu.make_async_copy(
                row_hbm.at[pl.ds(base, G)], rowg_v.at[buf], seme).wait()
            pltpu.make_async_copy(
                col_hbm.at[pl.ds(base, G)], colg_v.at[buf], seme).wait()
            pltpu.make_async_copy(
                w_hbm.at[pl.ds(base, G)], weg_v.at[buf], seme).wait()

        def start_gather(eb, jj, xb, sem):
            pltpu.make_async_copy(
                x_hbm.at[rowg_v.at[eb, jj]], xbuf_v.at[xb], sem).start()

        def wait_gather(eb, jj, xb, sem):
            pltpu.make_async_copy(
                x_hbm.at[rowg_v.at[eb, jj]], xbuf_v.at[xb], sem).wait()

        def process(eb, jj, xb):
            for g in range(CH // L):
                r16 = rowg_v[eb, jj, pl.ds(g * L, L)]
                c16 = colg_v[eb, jj, pl.ds(g * L, L)]
                dr = plsc.load_gather(dinv_v, [r16])
                dc = plsc.load_gather(dinv_v, [c16])
                wch_v[pl.ds(g * L, L)] = dr * dc * weg_v[eb, jj, pl.ds(g * L, L)]

            def scale_row(e, _):
                ws = plsc.load_gather(wch_v, [jnp.full((L,), e, jnp.int32)])
                for g in range(D // L):
                    xbuf_v[xb, e, pl.ds(g * L, L)] = (
                        xbuf_v[xb, e, pl.ds(g * L, L)] * ws)
                return 0
            lax.fori_loop(0, CH, scale_row, 0)
            pltpu.sync_copy(xbuf_v.at[xb],
                            agg_sh.at[colg_v.at[eb, jj]], add=True)

        def main_group(g, eb):
            wait_edge_group(g, eb)
            start_gather(eb, 0, 0, semg0)
            start_gather(eb, 1, 1, semg1)
            for jj in range(G):
                xb = jj % 2
                sem = semg0 if xb == 0 else semg1
                wait_gather(eb, jj, xb, sem)
                process(eb, jj, xb)
                if jj + 2 < G:
                    start_gather(eb, jj + 2, xb, sem)

            @pl.when(g + 2 < NG)
            def _():
                start_edge_group(g + 2, eb)

        start_edge_group(0, 0)
        start_edge_group(1, 1)

        def main_pair(i, _):
            main_group(i * 2, 0)
            main_group(i * 2 + 1, 1)
            return 0
        lax.fori_loop(0, NG // 2, main_pair, 0)

        plsc.subcore_barrier()
        pltpu.sync_copy(agg_sh.at[pl.ds(sid * NPT, NPT)],
                        out_hbm.at[cid, pl.ds(sid * NPT, NPT)])

    return body(x, colp, wp, rowp)


def _tc_body(x_ref, p0_ref, p1_ref, w_ref, b_ref, o_ref):
    h = x_ref[...] + p0_ref[...] + p1_ref[...]
    o_ref[...] = lax.dot_general(
        h, w_ref[...], (((1,), (1,)), ((), ())),
        preferred_element_type=jnp.float32) + b_ref[...]


def _tc_final(xp, p0, p1, W, b2):
    blk = 512
    grid = (NPAD // blk,)
    return pl.pallas_call(
        _tc_body,
        grid=grid,
        in_specs=[
            pl.BlockSpec((blk, D), lambda i: (i, 0)),
            pl.BlockSpec((blk, D), lambda i: (i, 0)),
            pl.BlockSpec((blk, D), lambda i: (i, 0)),
            pl.BlockSpec((D, D), lambda i: (0, 0)),
            pl.BlockSpec((1, D), lambda i: (0, 0)),
        ],
        out_specs=pl.BlockSpec((blk, D), lambda i: (i, 0)),
        out_shape=jax.ShapeDtypeStruct((NPAD, D), jnp.float32),
    )(xp, p0, p1, W, b2)


def kernel(x, edge_index, weight, W, b):
    row = edge_index[0]
    col = edge_index[1]
    pad = EPAD - E
    rowp = jnp.pad(row, (0, pad)).reshape(NCHUNK, CH)
    colp = jnp.pad(col, (0, pad)).reshape(NCHUNK, CH)
    wp = jnp.pad(weight, (0, pad)).reshape(NCHUNK, CH)

    partials = _sc_aggregate(x, colp, wp, rowp)

    xp = jnp.pad(x, ((0, NPAD - N), (0, 0)))
    out = _tc_final(xp, partials[0], partials[1], W, b.reshape(1, D))
    return out[:N]


# speedup vs baseline: 10.9718x; 1.2778x over previous
import functools

import numpy as np

import jax
import jax.numpy as jnp
from jax import lax
from jax.experimental import pallas as pl
from jax.experimental.pallas import tpu as pltpu, tpu_sc as plsc

N = 10000
E = 320000
D = 128

NC, NS, L = 2, 16, 16
NW = NC * NS
CH = 128
CPW = 80
EPAD = NW * CPW * CH
NCHUNK = NW * CPW
CPT_DEG = NCHUNK // NS
NPAD = 10240
NPT = NPAD // NS
G = 4
NG_DEG = CPT_DEG // G
NG = CPW // G
DW = D // 2

_PERM = np.concatenate(
    [g * 32 + np.concatenate([np.arange(0, 32, 2), np.arange(1, 32, 2)])
     for g in range(D // 32)]
).astype(np.int32)


def _rsqrt_vec(d):
    i = plsc.bitcast(d, jnp.int32)
    i = jnp.full((L,), 0x5F3759DF, dtype=jnp.int32) - lax.shift_right_logical(i, 1)
    y = plsc.bitcast(i, jnp.float32)
    half_d = d * 0.5
    for _ in range(3):
        y = y * (1.5 - half_d * y * y)
    return jnp.where(d > 0.0, y, 0.0)


def _sc_aggregate(xw, colp, wp, rowp):
    mesh = plsc.VectorSubcoreMesh(core_axis_name="c", subcore_axis_name="s")

    @functools.partial(
        pl.kernel,
        mesh=mesh,
        out_type=jax.ShapeDtypeStruct((NC, NPAD, D), jnp.float32),
        scratch_types=[
            pltpu.VMEM((NPAD,), jnp.float32),
            pltpu.VMEM((CH,), jnp.float32),
            pltpu.VMEM((2, G, CH), jnp.int32),
            pltpu.VMEM((2, G, CH), jnp.int32),
            pltpu.VMEM((2, G, CH), jnp.float32),
            pltpu.VMEM((2, CH, DW), jnp.int32),
            pltpu.VMEM((CH, D), jnp.float32),
            pltpu.VMEM_SHARED((NPAD, D), jnp.float32),
            pltpu.VMEM_SHARED((NPAD,), jnp.float32),
            pltpu.VMEM_SHARED((NPAD,), jnp.float32),
            pltpu.SemaphoreType.DMA,
            pltpu.SemaphoreType.DMA,
            pltpu.SemaphoreType.DMA,
        ],
        compiler_params=pltpu.CompilerParams(
            needs_layout_passes=False, use_tc_tiling_on_sc=False),
    )
    def body(x_hbm, col_hbm, w_hbm, row_hbm, out_hbm,
             dinv_v, wch_v, rowg_v, colg_v, weg_v, xbuf_v, sc_v,
             agg_sh, deg_sh, dinv_sh, semg0, semg1, seme):
        cid = lax.axis_index("c")
        sid = lax.axis_index("s")
        wid = sid * NC + cid

        def zero_sc(r, _):
            for g in range(D // L):
                sc_v[r, pl.ds(g * L, L)] = jnp.zeros((L,), jnp.float32)
            return 0
        lax.fori_loop(0, CH, zero_sc, 0)
        for k in range(NPT // CH):
            pltpu.sync_copy(sc_v, agg_sh.at[pl.ds(sid * NPT + k * CH, CH)])
        for g in range(CH // L):
            wch_v[pl.ds(g * L, L)] = jnp.zeros((L,), jnp.float32)
        for k in range(NPT // CH):
            pltpu.sync_copy(wch_v, deg_sh.at[pl.ds(sid * NPT + k * CH, CH)])
        plsc.subcore_barrier()

        def start_deg_group(g, buf):
            base = sid * CPT_DEG + g * G
            pltpu.make_async_copy(
                col_hbm.at[pl.ds(base, G)], colg_v.at[buf], seme).start()
            pltpu.make_async_copy(
                w_hbm.at[pl.ds(base, G)], weg_v.at[buf], seme).start()

        def wait_deg_group(g, buf):
            base = sid * CPT_DEG + g * G
            pltpu.make_async_copy(
                col_hbm.at[pl.ds(base, G)], colg_v.at[buf], seme).wait()
            pltpu.make_async_copy(
                w_hbm.at[pl.ds(base, G)], weg_v.at[buf], seme).wait()

        def deg_group(g, buf):
            wait_deg_group(g, buf)
            for jj in range(G):
                pltpu.sync_copy(weg_v.at[buf, jj],
                                deg_sh.at[colg_v.at[buf, jj]], add=True)

            @pl.when(g + 2 < NG_DEG)
            def _():
                start_deg_group(g + 2, buf)

        start_deg_group(0, 0)
        start_deg_group(1, 1)

        def deg_pair(i, _):
            deg_group(i * 2, 0)
            deg_group(i * 2 + 1, 1)
            return 0
        lax.fori_loop(0, NG_DEG // 2, deg_pair, 0)
        plsc.subcore_barrier()

        def dinv_piece(k, _):
            pltpu.sync_copy(deg_sh.at[pl.ds(sid * NPT + k * CH, CH)], wch_v)
            for g in range(CH // L):
                wch_v[pl.ds(g * L, L)] = _rsqrt_vec(wch_v[pl.ds(g * L, L)])
            pltpu.sync_copy(wch_v, dinv_sh.at[pl.ds(sid * NPT + k * CH, CH)])
            return 0
        lax.fori_loop(0, NPT // CH, dinv_piece, 0)
        plsc.subcore_barrier()
        pltpu.sync_copy(dinv_sh, dinv_v)

        def start_edge_group(g, buf):
            base = wid * CPW + g * G
            pltpu.make_async_copy(
                row_hbm.at[pl.ds(base, G)], rowg_v.at[buf], seme).start()
            pltpu.make_async_copy(
                col_hbm.at[pl.ds(base, G)], colg_v.at[buf], seme).start()
            pltpu.make_async_copy(
                w_hbm.at[pl.ds(base, G)], weg_v.at[buf], seme).start()

        def wait_edge_group(g, buf):
            base = wid * CPW + g * G
            pltpu.make_async_copy(
                row_hbm.at[pl.ds(base, G)], rowg_v.at[buf], seme).wait()
            pltpu.make_async_copy(
                col_hbm.at[pl.ds(base, G)], colg_v.at[buf], seme).wait()
            pltpu.make_async_copy(
                w_hbm.at[pl.ds(base, G)], weg_v.at[buf], seme).wait()

        def start_gather(eb, jj, xb, sem):
            pltpu.make_async_copy(
                x_hbm.at[rowg_v.at[eb, jj]], xbuf_v.at[xb], sem).start()

        def wait_gather(eb, jj, xb, sem):
            pltpu.make_async_copy(
                x_hbm.at[rowg_v.at[eb, jj]], xbuf_v.at[xb], sem).wait()

        def process(eb, jj, xb):
            for g in range(CH // L):
                r16 = rowg_v[eb, jj, pl.ds(g * L, L)]
                c16 = colg_v[eb, jj, pl.ds(g * L, L)]
                dr = plsc.load_gather(dinv_v, [r16])
                dc = plsc.load_gather(dinv_v, [c16])
                wch_v[pl.ds(g * L, L)] = dr * dc * weg_v[eb, jj, pl.ds(g * L, L)]

            def scale_row(e, _):
                ws = plsc.load_gather(wch_v, [jnp.full((L,), e, jnp.int32)])
                for g in range(D // 32):
                    pk = xbuf_v[xb, e, pl.ds(g * L, L)]
                    bf = plsc.bitcast(pk, jnp.bfloat16)
                    lo, hi = plsc.unpack(bf, format=plsc.PackFormat.INTERLEAVED)
                    sc_v[e, pl.ds(g * 32, L)] = lo * ws
                    sc_v[e, pl.ds(g * 32 + L, L)] = hi * ws
                return 0
            lax.fori_loop(0, CH, scale_row, 0)
            pltpu.sync_copy(sc_v, agg_sh.at[colg_v.at[eb, jj]], add=True)

        def main_group(g, eb):
            wait_edge_group(g, eb)
            start_gather(eb, 0, 0, semg0)
            start_gather(eb, 1, 1, semg1)
            for jj in range(G):
                xb = jj % 2
                sem = semg0 if xb == 0 else semg1
                wait_gather(eb, jj, xb, sem)
                process(eb, jj, xb)
                if jj + 2 < G:
                    start_gather(eb, jj + 2, xb, sem)

            @pl.when(g + 2 < NG)
            def _():
                start_edge_group(g + 2, eb)

        start_edge_group(0, 0)
        start_edge_group(1, 1)

        def main_pair(i, _):
            main_group(i * 2, 0)
            main_group(i * 2 + 1, 1)
            return 0
        lax.fori_loop(0, NG // 2, main_pair, 0)

        plsc.subcore_barrier()
        pltpu.sync_copy(agg_sh.at[pl.ds(sid * NPT, NPT)],
                        out_hbm.at[cid, pl.ds(sid * NPT, NPT)])

    return body(xw, colp, wp, rowp)


def _tc_body(x_ref, p0_ref, p1_ref, w_ref, b_ref, o_ref):
    h = x_ref[...] + p0_ref[...] + p1_ref[...]
    o_ref[...] = lax.dot_general(
        h, w_ref[...], (((1,), (1,)), ((), ())),
        preferred_element_type=jnp.float32) + b_ref[...]


def _tc_final(xp, p0, p1, Wp, b2):
    blk = 512
    grid = (NPAD // blk,)
    return pl.pallas_call(
        _tc_body,
        grid=grid,
        in_specs=[
            pl.BlockSpec((blk, D), lambda i: (i, 0)),
            pl.BlockSpec((blk, D), lambda i: (i, 0)),
            pl.BlockSpec((blk, D), lambda i: (i, 0)),
            pl.BlockSpec((D, D), lambda i: (0, 0)),
            pl.BlockSpec((1, D), lambda i: (0, 0)),
        ],
        out_specs=pl.BlockSpec((blk, D), lambda i: (i, 0)),
        out_shape=jax.ShapeDtypeStruct((NPAD, D), jnp.float32),
    )(xp, p0, p1, Wp, b2)


def kernel(x, edge_index, weight, W, b):
    row = edge_index[0]
    col = edge_index[1]
    pad = EPAD - E
    rowp = jnp.pad(row, (0, pad)).reshape(NCHUNK, CH)
    colp = jnp.pad(col, (0, pad)).reshape(NCHUNK, CH)
    wp = jnp.pad(weight, (0, pad)).reshape(NCHUNK, CH)
    xw = lax.bitcast_convert_type(
        x.astype(jnp.bfloat16).reshape(N, DW, 2), jnp.int32)

    partials = _sc_aggregate(xw, colp, wp, rowp)

    perm = jnp.asarray(_PERM)
    xp = jnp.pad(x[:, perm], ((0, NPAD - N), (0, 0)))
    out = _tc_final(xp, partials[0], partials[1], W[:, perm], b.reshape(1, D))
    return out[:N]


# speedup vs baseline: 11.3138x; 1.0312x over previous
import functools

import numpy as np

import jax
import jax.numpy as jnp
from jax import lax
from jax.experimental import pallas as pl
from jax.experimental.pallas import tpu as pltpu, tpu_sc as plsc

N = 10000
E = 320000
D = 128

NC, NS, L = 2, 16, 16
NW = NC * NS
CH = 128
CPW = 80
EPAD = NW * CPW * CH
NCHUNK = NW * CPW
CPT_DEG = NCHUNK // NS
NPAD = 10240
NPT = NPAD // NS
G = 4
NG_DEG = CPT_DEG // G
NG = CPW // G
DW = D // 2

def _rsqrt_vec(d):
    i = plsc.bitcast(d, jnp.int32)
    i = jnp.full((L,), 0x5F3759DF, dtype=jnp.int32) - lax.shift_right_logical(i, 1)
    y = plsc.bitcast(i, jnp.float32)
    half_d = d * 0.5
    for _ in range(3):
        y = y * (1.5 - half_d * y * y)
    return jnp.where(d > 0.0, y, 0.0)


def _sc_aggregate(xw, colp, wp, rowp):
    mesh = plsc.VectorSubcoreMesh(core_axis_name="c", subcore_axis_name="s")

    @functools.partial(
        pl.kernel,
        mesh=mesh,
        out_type=jax.ShapeDtypeStruct((NC, NPAD, D), jnp.float32),
        scratch_types=[
            pltpu.VMEM((NPAD,), jnp.float32),
            pltpu.VMEM((CH,), jnp.float32),
            pltpu.VMEM((2, G, CH), jnp.int32),
            pltpu.VMEM((2, G, CH), jnp.int32),
            pltpu.VMEM((2, G, CH), jnp.float32),
            pltpu.VMEM((2, CH, DW), jnp.int32),
            pltpu.VMEM((CH, D), jnp.float32),
            pltpu.VMEM_SHARED((NPAD, D), jnp.float32),
            pltpu.VMEM_SHARED((NPAD,), jnp.float32),
            pltpu.VMEM_SHARED((NPAD,), jnp.float32),
            pltpu.SemaphoreType.DMA,
            pltpu.SemaphoreType.DMA,
            pltpu.SemaphoreType.DMA,
        ],
        compiler_params=pltpu.CompilerParams(
            needs_layout_passes=False, use_tc_tiling_on_sc=False),
    )
    def body(x_hbm, col_hbm, w_hbm, row_hbm, out_hbm,
             dinv_v, wch_v, rowg_v, colg_v, weg_v, xbuf_v, sc_v,
             agg_sh, deg_sh, dinv_sh, semg0, semg1, seme):
        cid = lax.axis_index("c")
        sid = lax.axis_index("s")
        wid = sid * NC + cid

        def zero_sc(r, _):
            for g in range(D // L):
                sc_v[r, pl.ds(g * L, L)] = jnp.zeros((L,), jnp.float32)
            return 0
        lax.fori_loop(0, CH, zero_sc, 0)
        for k in range(NPT // CH):
            pltpu.sync_copy(sc_v, agg_sh.at[pl.ds(sid * NPT + k * CH, CH)])
        for g in range(CH // L):
            wch_v[pl.ds(g * L, L)] = jnp.zeros((L,), jnp.float32)
        for k in range(NPT // CH):
            pltpu.sync_copy(wch_v, deg_sh.at[pl.ds(sid * NPT + k * CH, CH)])
        plsc.subcore_barrier()

        def start_deg_group(g, buf):
            base = sid * CPT_DEG + g * G
            pltpu.make_async_copy(
                col_hbm.at[pl.ds(base, G)], colg_v.at[buf], seme).start()
            pltpu.make_async_copy(
                w_hbm.at[pl.ds(base, G)], weg_v.at[buf], seme).start()

        def wait_deg_group(g, buf):
            base = sid * CPT_DEG + g * G
            pltpu.make_async_copy(
                col_hbm.at[pl.ds(base, G)], colg_v.at[buf], seme).wait()
            pltpu.make_async_copy(
                w_hbm.at[pl.ds(base, G)], weg_v.at[buf], seme).wait()

        def deg_group(g, buf):
            wait_deg_group(g, buf)
            for jj in range(G):
                pltpu.sync_copy(weg_v.at[buf, jj],
                                deg_sh.at[colg_v.at[buf, jj]], add=True)

            @pl.when(g + 2 < NG_DEG)
            def _():
                start_deg_group(g + 2, buf)

        start_deg_group(0, 0)
        start_deg_group(1, 1)

        def deg_pair(i, _):
            deg_group(i * 2, 0)
            deg_group(i * 2 + 1, 1)
            return 0
        lax.fori_loop(0, NG_DEG // 2, deg_pair, 0)
        plsc.subcore_barrier()

        def dinv_piece(k, _):
            pltpu.sync_copy(deg_sh.at[pl.ds(sid * NPT + k * CH, CH)], wch_v)
            for g in range(CH // L):
                wch_v[pl.ds(g * L, L)] = _rsqrt_vec(wch_v[pl.ds(g * L, L)])
            pltpu.sync_copy(wch_v, dinv_sh.at[pl.ds(sid * NPT + k * CH, CH)])
            return 0
        lax.fori_loop(0, NPT // CH, dinv_piece, 0)
        plsc.subcore_barrier()
        pltpu.sync_copy(dinv_sh, dinv_v)

        def start_edge_group(g, buf):
            base = wid * CPW + g * G
            pltpu.make_async_copy(
                row_hbm.at[pl.ds(base, G)], rowg_v.at[buf], seme).start()
            pltpu.make_async_copy(
                col_hbm.at[pl.ds(base, G)], colg_v.at[buf], seme).start()
            pltpu.make_async_copy(
                w_hbm.at[pl.ds(base, G)], weg_v.at[buf], seme).start()

        def wait_edge_group(g, buf):
            base = wid * CPW + g * G
            pltpu.make_async_copy(
                row_hbm.at[pl.ds(base, G)], rowg_v.at[buf], seme).wait()
            pltpu.make_async_copy(
                col_hbm.at[pl.ds(base, G)], colg_v.at[buf], seme).wait()
            pltpu.make_async_copy(
                w_hbm.at[pl.ds(base, G)], weg_v.at[buf], seme).wait()

        def start_gather(eb, jj, xb, sem):
            pltpu.make_async_copy(
                x_hbm.at[rowg_v.at[eb, jj]], xbuf_v.at[xb], sem).start()

        def wait_gather(eb, jj, xb, sem):
            pltpu.make_async_copy(
                x_hbm.at[rowg_v.at[eb, jj]], xbuf_v.at[xb], sem).wait()

        def process(eb, jj, xb):
            for g in range(CH // L):
                r16 = rowg_v[eb, jj, pl.ds(g * L, L)]
                c16 = colg_v[eb, jj, pl.ds(g * L, L)]
                dr = plsc.load_gather(dinv_v, [r16])
                dc = plsc.load_gather(dinv_v, [c16])
                wch_v[pl.ds(g * L, L)] = dr * dc * weg_v[eb, jj, pl.ds(g * L, L)]

            def scale_row(e, _):
                e16 = jnp.full((L,), e, jnp.int32)
                ws = plsc.load_gather(wch_v, [e16])
                even = lax.iota(jnp.int32, L) * 2
                odd = even + 1
                for g in range(D // 32):
                    pk = xbuf_v[xb, e, pl.ds(g * L, L)]
                    bf = plsc.bitcast(pk, jnp.bfloat16)
                    lo, hi = plsc.unpack(bf, format=plsc.PackFormat.INTERLEAVED)
                    plsc.store_scatter(sc_v, [e16, g * 32 + even], lo * ws)
                    plsc.store_scatter(sc_v, [e16, g * 32 + odd], hi * ws)
                return 0
            lax.fori_loop(0, CH, scale_row, 0)
            pltpu.sync_copy(sc_v, agg_sh.at[colg_v.at[eb, jj]], add=True)

        def main_group(g, eb):
            wait_edge_group(g, eb)
            start_gather(eb, 0, 0, semg0)
            start_gather(eb, 1, 1, semg1)
            for jj in range(G):
                xb = jj % 2
                sem = semg0 if xb == 0 else semg1
                wait_gather(eb, jj, xb, sem)
                process(eb, jj, xb)
                if jj + 2 < G:
                    start_gather(eb, jj + 2, xb, sem)

            @pl.when(g + 2 < NG)
            def _():
                start_edge_group(g + 2, eb)

        start_edge_group(0, 0)
        start_edge_group(1, 1)

        def main_pair(i, _):
            main_group(i * 2, 0)
            main_group(i * 2 + 1, 1)
            return 0
        lax.fori_loop(0, NG // 2, main_pair, 0)

        plsc.subcore_barrier()
        pltpu.sync_copy(agg_sh.at[pl.ds(sid * NPT, NPT)],
                        out_hbm.at[cid, pl.ds(sid * NPT, NPT)])

    return body(xw, colp, wp, rowp)


def _tc_body(x_ref, p0_ref, p1_ref, w_ref, b_ref, o_ref):
    h = x_ref[...] + p0_ref[0] + p1_ref[0]
    o_ref[...] = lax.dot_general(
        h, w_ref[...], (((1,), (1,)), ((), ())),
        preferred_element_type=jnp.float32) + b_ref[...]


def _tc_final(xp, partials, Wp, b2):
    blk = 400
    grid = (N // blk,)
    return pl.pallas_call(
        _tc_body,
        grid=grid,
        in_specs=[
            pl.BlockSpec((blk, D), lambda i: (i, 0)),
            pl.BlockSpec((1, blk, D), lambda i: (0, i, 0)),
            pl.BlockSpec((1, blk, D), lambda i: (1, i, 0)),
            pl.BlockSpec((D, D), lambda i: (0, 0)),
            pl.BlockSpec((1, D), lambda i: (0, 0)),
        ],
        out_specs=pl.BlockSpec((blk, D), lambda i: (i, 0)),
        out_shape=jax.ShapeDtypeStruct((N, D), jnp.float32),
    )(xp, partials, partials, Wp, b2)


def kernel(x, edge_index, weight, W, b):
    row = edge_index[0]
    col = edge_index[1]
    pad = EPAD - E
    rowp = jnp.pad(row, (0, pad)).reshape(NCHUNK, CH)
    colp = jnp.pad(col, (0, pad)).reshape(NCHUNK, CH)
    wp = jnp.pad(weight, (0, pad)).reshape(NCHUNK, CH)
    xw = lax.bitcast_convert_type(
        x.astype(jnp.bfloat16).reshape(N, DW, 2), jnp.int32)

    partials = _sc_aggregate(xw, colp, wp, rowp)
    return _tc_final(x, partials, W, b.reshape(1, D))


# speedup vs baseline: 13.3695x; 1.1817x over previous
import functools

import jax
import jax.numpy as jnp
from jax import lax
from jax.experimental import pallas as pl
from jax.experimental.pallas import tpu as pltpu, tpu_sc as plsc

N = 10000
E = 320000
D = 128

NC, NS, L = 2, 16, 16
NW = NC * NS
CH = 64
CPW = 160
EPAD = NW * CPW * CH
NCHUNK = NW * CPW
CPT_DEG = NCHUNK // NS
NPAD = 10240
NPT = NPAD // NS
G = 4
NG_DEG = CPT_DEG // G
NG = CPW // G
DW = D // 2


def _rsqrt_vec(d):
    i = plsc.bitcast(d, jnp.int32)
    i = jnp.full((L,), 0x5F3759DF, dtype=jnp.int32) - lax.shift_right_logical(i, 1)
    y = plsc.bitcast(i, jnp.float32)
    half_d = d * 0.5
    for _ in range(3):
        y = y * (1.5 - half_d * y * y)
    return jnp.where(d > 0.0, y, 0.0)


def _sc_aggregate(xw, colp, wp, rowp):
    mesh = plsc.VectorSubcoreMesh(core_axis_name="c", subcore_axis_name="s")

    @functools.partial(
        pl.kernel,
        mesh=mesh,
        out_type=jax.ShapeDtypeStruct((NC, NPAD, D), jnp.float32),
        scratch_types=[
            pltpu.VMEM((NPAD,), jnp.float32),
            pltpu.VMEM((CH,), jnp.float32),
            pltpu.VMEM((3, G, CH), jnp.int32),
            pltpu.VMEM((3, G, CH), jnp.int32),
            pltpu.VMEM((3, G, CH), jnp.float32),
            pltpu.VMEM((G, CH, DW), jnp.int32),
            pltpu.VMEM((CH, D), jnp.float32),
            pltpu.VMEM_SHARED((NPAD, D), jnp.float32),
            pltpu.VMEM_SHARED((NPAD,), jnp.float32),
            pltpu.VMEM_SHARED((NPAD,), jnp.float32),
            pltpu.SemaphoreType.DMA,
            pltpu.SemaphoreType.DMA,
            pltpu.SemaphoreType.DMA,
            pltpu.SemaphoreType.DMA,
            pltpu.SemaphoreType.DMA,
        ],
        compiler_params=pltpu.CompilerParams(
            needs_layout_passes=False, use_tc_tiling_on_sc=False),
    )
    def body(x_hbm, col_hbm, w_hbm, row_hbm, out_hbm,
             dinv_v, wch_v, rowg_v, colg_v, weg_v, xbuf_v, sc_v,
             agg_sh, deg_sh, dinv_sh, semg0, semg1, semg2, semg3, seme):
        cid = lax.axis_index("c")
        sid = lax.axis_index("s")
        wid = sid * NC + cid
        gsems = [semg0, semg1, semg2, semg3]

        def zero_sc(r, _):
            for g in range(D // L):
                sc_v[r, pl.ds(g * L, L)] = jnp.zeros((L,), jnp.float32)
            return 0
        lax.fori_loop(0, CH, zero_sc, 0)
        for k in range(NPT // CH):
            pltpu.sync_copy(sc_v, agg_sh.at[pl.ds(sid * NPT + k * CH, CH)])
        for g in range(CH // L):
            wch_v[pl.ds(g * L, L)] = jnp.zeros((L,), jnp.float32)
        for k in range(NPT // CH):
            pltpu.sync_copy(wch_v, deg_sh.at[pl.ds(sid * NPT + k * CH, CH)])
        plsc.subcore_barrier()

        def start_deg_group(g, buf):
            base = sid * CPT_DEG + g * G
            pltpu.make_async_copy(
                col_hbm.at[pl.ds(base, G)], colg_v.at[buf], seme).start()
            pltpu.make_async_copy(
                w_hbm.at[pl.ds(base, G)], weg_v.at[buf], seme).start()

        def wait_deg_group(g, buf):
            base = sid * CPT_DEG + g * G
            pltpu.make_async_copy(
                col_hbm.at[pl.ds(base, G)], colg_v.at[buf], seme).wait()
            pltpu.make_async_copy(
                w_hbm.at[pl.ds(base, G)], weg_v.at[buf], seme).wait()

        def deg_group(g, buf):
            wait_deg_group(g, buf)
            for jj in range(G):
                pltpu.sync_copy(weg_v.at[buf, jj],
                                deg_sh.at[colg_v.at[buf, jj]], add=True)

            @pl.when(g + 2 < NG_DEG)
            def _():
                start_deg_group(g + 2, buf)

        start_deg_group(0, 0)
        start_deg_group(1, 1)

        def deg_pair(i, _):
            deg_group(i * 2, 0)
            deg_group(i * 2 + 1, 1)
            return 0
        lax.fori_loop(0, NG_DEG // 2, deg_pair, 0)
        plsc.subcore_barrier()

        def dinv_piece(k, _):
            pltpu.sync_copy(deg_sh.at[pl.ds(sid * NPT + k * CH, CH)], wch_v)
            for g in range(CH // L):
                wch_v[pl.ds(g * L, L)] = _rsqrt_vec(wch_v[pl.ds(g * L, L)])
            pltpu.sync_copy(wch_v, dinv_sh.at[pl.ds(sid * NPT + k * CH, CH)])
            return 0
        lax.fori_loop(0, NPT // CH, dinv_piece, 0)
        plsc.subcore_barrier()
        pltpu.sync_copy(dinv_sh, dinv_v)

        def start_edge_group(g, buf):
            base = wid * CPW + g * G
            pltpu.make_async_copy(
                row_hbm.at[pl.ds(base, G)], rowg_v.at[buf], seme).start()
            pltpu.make_async_copy(
                col_hbm.at[pl.ds(base, G)], colg_v.at[buf], seme).start()
            pltpu.make_async_copy(
                w_hbm.at[pl.ds(base, G)], weg_v.at[buf], seme).start()

        def wait_edge_group(g, buf):
            base = wid * CPW + g * G
            pltpu.make_async_copy(
                row_hbm.at[pl.ds(base, G)], rowg_v.at[buf], seme).wait()
            pltpu.make_async_copy(
                col_hbm.at[pl.ds(base, G)], colg_v.at[buf], seme).wait()
            pltpu.make_async_copy(
                w_hbm.at[pl.ds(base, G)], weg_v.at[buf], seme).wait()

        def start_gather(eb, jj):
            pltpu.make_async_copy(
                x_hbm.at[rowg_v.at[eb, jj]], xbuf_v.at[jj], gsems[jj]).start()

        def wait_gather(eb, jj):
            pltpu.make_async_copy(
                x_hbm.at[rowg_v.at[eb, jj]], xbuf_v.at[jj], gsems[jj]).wait()

        def process(eb, jj):
            for g in range(CH // L):
                r16 = rowg_v[eb, jj, pl.ds(g * L, L)]
                c16 = colg_v[eb, jj, pl.ds(g * L, L)]
                dr = plsc.load_gather(dinv_v, [r16])
                dc = plsc.load_gather(dinv_v, [c16])
                wch_v[pl.ds(g * L, L)] = dr * dc * weg_v[eb, jj, pl.ds(g * L, L)]

            def scale_row(e, _):
                e16 = jnp.full((L,), e, jnp.int32)
                ws = plsc.load_gather(wch_v, [e16])
                even = lax.iota(jnp.int32, L) * 2
                odd = even + 1
                for g in range(D // 32):
                    pk = xbuf_v[jj, e, pl.ds(g * L, L)]
                    bf = plsc.bitcast(pk, jnp.bfloat16)
                    lo, hi = plsc.unpack(bf, format=plsc.PackFormat.INTERLEAVED)
                    plsc.store_scatter(sc_v, [e16, g * 32 + even], lo * ws)
                    plsc.store_scatter(sc_v, [e16, g * 32 + odd], hi * ws)
                return 0
            lax.fori_loop(0, CH, scale_row, 0)
            pltpu.sync_copy(sc_v, agg_sh.at[colg_v.at[eb, jj]], add=True)

        def main_group(g, eb, ebn):
            @pl.when(g + 2 < NG)
            def _():
                start_edge_group(g + 2, (eb + 2) % 3)

            @pl.when(g + 1 < NG)
            def _():
                wait_edge_group(g + 1, ebn)
            for jj in range(G):
                wait_gather(eb, jj)
                process(eb, jj)

                @pl.when(g + 1 < NG)
                def _():
                    start_gather(ebn, jj)

        start_edge_group(0, 0)
        start_edge_group(1, 1)
        wait_edge_group(0, 0)
        for jj in range(G):
            start_gather(0, jj)

        def main_tri(i, _):
            main_group(i * 3, 0, 1)
            main_group(i * 3 + 1, 1, 2)
            main_group(i * 3 + 2, 2, 0)
            return 0
        lax.fori_loop(0, NG // 3, main_tri, 0)
        main_group(NG - 1, (NG - 1) % 3, NG % 3)

        plsc.subcore_barrier()
        pltpu.sync_copy(agg_sh.at[pl.ds(sid * NPT, NPT)],
                        out_hbm.at[cid, pl.ds(sid * NPT, NPT)])

    return body(xw, colp, wp, rowp)


def _tc_body(x_ref, p0_ref, p1_ref, w_ref, b_ref, o_ref):
    h = x_ref[...] + p0_ref[0] + p1_ref[0]
    o_ref[...] = lax.dot_general(
        h, w_ref[...], (((1,), (1,)), ((), ())),
        preferred_element_type=jnp.float32) + b_ref[...]


def _tc_final(xp, partials, Wp, b2):
    blk = 400
    grid = (N // blk,)
    return pl.pallas_call(
        _tc_body,
        grid=grid,
        in_specs=[
            pl.BlockSpec((blk, D), lambda i: (i, 0)),
            pl.BlockSpec((1, blk, D), lambda i: (0, i, 0)),
            pl.BlockSpec((1, blk, D), lambda i: (1, i, 0)),
            pl.BlockSpec((D, D), lambda i: (0, 0)),
            pl.BlockSpec((1, D), lambda i: (0, 0)),
        ],
        out_specs=pl.BlockSpec((blk, D), lambda i: (i, 0)),
        out_shape=jax.ShapeDtypeStruct((N, D), jnp.float32),
    )(xp, partials, partials, Wp, b2)


def kernel(x, edge_index, weight, W, b):
    row = edge_index[0]
    col = edge_index[1]
    pad = EPAD - E
    rowp = jnp.pad(row, (0, pad)).reshape(NCHUNK, CH)
    colp = jnp.pad(col, (0, pad)).reshape(NCHUNK, CH)
    wp = jnp.pad(weight, (0, pad)).reshape(NCHUNK, CH)
    xw = lax.bitcast_convert_type(
        x.astype(jnp.bfloat16).reshape(N, DW, 2), jnp.int32)

    partials = _sc_aggregate(xw, colp, wp, rowp)
    return _tc_final(x, partials, W, b.reshape(1, D))


# speedup vs baseline: 13.4803x; 1.0083x over previous
import functools

import jax
import jax.numpy as jnp
from jax import lax
from jax.experimental import pallas as pl
from jax.experimental.pallas import tpu as pltpu, tpu_sc as plsc

N = 10000
E = 320000
D = 128

NC, NS, L = 2, 16, 16
NW = NC * NS
CH = 64
CPW = 160
EPAD = NW * CPW * CH
NCHUNK = NW * CPW
CPT_DEG = NCHUNK // NS
NPAD = 10240
NPT = NPAD // NS
G = 8
NG_DEG = CPT_DEG // G
NG = CPW // G
DW = D // 2


def _rsqrt_vec(d):
    i = plsc.bitcast(d, jnp.int32)
    i = jnp.full((L,), 0x5F3759DF, dtype=jnp.int32) - lax.shift_right_logical(i, 1)
    y = plsc.bitcast(i, jnp.float32)
    half_d = d * 0.5
    for _ in range(3):
        y = y * (1.5 - half_d * y * y)
    return jnp.where(d > 0.0, y, 0.0)


def _sc_aggregate(xw, colp, wp, rowp):
    mesh = plsc.VectorSubcoreMesh(core_axis_name="c", subcore_axis_name="s")

    @functools.partial(
        pl.kernel,
        mesh=mesh,
        out_type=jax.ShapeDtypeStruct((NC, NPAD, D), jnp.float32),
        scratch_types=[
            pltpu.VMEM((NPAD,), jnp.float32),
            pltpu.VMEM((CH,), jnp.float32),
            pltpu.VMEM((3, G, CH), jnp.int32),
            pltpu.VMEM((3, G, CH), jnp.int32),
            pltpu.VMEM((3, G, CH), jnp.float32),
            pltpu.VMEM((4, CH, DW), jnp.int32),
            pltpu.VMEM((CH, D), jnp.float32),
            pltpu.VMEM_SHARED((NPAD, D), jnp.float32),
            pltpu.VMEM_SHARED((NPAD,), jnp.float32),
            pltpu.VMEM_SHARED((NPAD,), jnp.float32),
            pltpu.SemaphoreType.DMA,
            pltpu.SemaphoreType.DMA,
            pltpu.SemaphoreType.DMA,
            pltpu.SemaphoreType.DMA,
            pltpu.SemaphoreType.DMA,
        ],
        compiler_params=pltpu.CompilerParams(
            needs_layout_passes=False, use_tc_tiling_on_sc=False),
    )
    def body(x_hbm, col_hbm, w_hbm, row_hbm, out_hbm,
             dinv_v, wch_v, rowg_v, colg_v, weg_v, xbuf_v, sc_v,
             agg_sh, deg_sh, dinv_sh, semg0, semg1, semg2, semg3, seme):
        cid = lax.axis_index("c")
        sid = lax.axis_index("s")
        wid = sid * NC + cid
        gsems = [semg0, semg1, semg2, semg3]

        def start_edge_group(g, buf):
            base = wid * CPW + g * G
            pltpu.make_async_copy(
                row_hbm.at[pl.ds(base, G)], rowg_v.at[buf], seme).start()
            pltpu.make_async_copy(
                col_hbm.at[pl.ds(base, G)], colg_v.at[buf], seme).start()
            pltpu.make_async_copy(
                w_hbm.at[pl.ds(base, G)], weg_v.at[buf], seme).start()

        def wait_edge_group(g, buf):
            base = wid * CPW + g * G
            pltpu.make_async_copy(
                row_hbm.at[pl.ds(base, G)], rowg_v.at[buf], seme).wait()
            pltpu.make_async_copy(
                col_hbm.at[pl.ds(base, G)], colg_v.at[buf], seme).wait()
            pltpu.make_async_copy(
                w_hbm.at[pl.ds(base, G)], weg_v.at[buf], seme).wait()

        def start_gather(eb, jj):
            pltpu.make_async_copy(
                x_hbm.at[rowg_v.at[eb, jj]], xbuf_v.at[jj % 4],
                gsems[jj % 4]).start()

        def wait_gather(eb, jj):
            pltpu.make_async_copy(
                x_hbm.at[rowg_v.at[eb, jj]], xbuf_v.at[jj % 4],
                gsems[jj % 4]).wait()

        start_edge_group(0, 2)

        def zero_sc(r, _):
            for g in range(D // L):
                sc_v[r, pl.ds(g * L, L)] = jnp.zeros((L,), jnp.float32)
            return 0
        lax.fori_loop(0, CH, zero_sc, 0)
        for k in range(NPT // CH):
            pltpu.sync_copy(sc_v, agg_sh.at[pl.ds(sid * NPT + k * CH, CH)])
        for g in range(CH // L):
            wch_v[pl.ds(g * L, L)] = jnp.zeros((L,), jnp.float32)
        for k in range(NPT // CH):
            pltpu.sync_copy(wch_v, deg_sh.at[pl.ds(sid * NPT + k * CH, CH)])
        plsc.subcore_barrier()

        def start_deg_group(g, buf):
            base = sid * CPT_DEG + g * G
            pltpu.make_async_copy(
                col_hbm.at[pl.ds(base, G)], colg_v.at[buf], seme).start()
            pltpu.make_async_copy(
                w_hbm.at[pl.ds(base, G)], weg_v.at[buf], seme).start()

        def wait_deg_group(g, buf):
            base = sid * CPT_DEG + g * G
            pltpu.make_async_copy(
                col_hbm.at[pl.ds(base, G)], colg_v.at[buf], seme).wait()
            pltpu.make_async_copy(
                w_hbm.at[pl.ds(base, G)], weg_v.at[buf], seme).wait()

        def deg_group(g, buf):
            wait_deg_group(g, buf)
            for jj in range(G):
                pltpu.sync_copy(weg_v.at[buf, jj],
                                deg_sh.at[colg_v.at[buf, jj]], add=True)

            @pl.when(g + 2 < NG_DEG)
            def _():
                start_deg_group(g + 2, buf)

        start_deg_group(0, 0)
        start_deg_group(1, 1)
        wait_edge_group(0, 2)
        for jj in range(4):
            start_gather(2, jj)

        def deg_pair(i, _):
            deg_group(i * 2, 0)
            deg_group(i * 2 + 1, 1)
            return 0
        lax.fori_loop(0, NG_DEG // 2, deg_pair, 0)
        start_edge_group(1, 0)
        plsc.subcore_barrier()

        def dinv_piece(k, _):
            pltpu.sync_copy(deg_sh.at[pl.ds(sid * NPT + k * CH, CH)], wch_v)
            for g in range(CH // L):
                wch_v[pl.ds(g * L, L)] = _rsqrt_vec(wch_v[pl.ds(g * L, L)])
            pltpu.sync_copy(wch_v, dinv_sh.at[pl.ds(sid * NPT + k * CH, CH)])
            return 0
        lax.fori_loop(0, NPT // CH, dinv_piece, 0)
        plsc.subcore_barrier()
        pltpu.sync_copy(dinv_sh, dinv_v)

        def process(eb, jj):
            for g in range(CH // L):
                r16 = rowg_v[eb, jj, pl.ds(g * L, L)]
                c16 = colg_v[eb, jj, pl.ds(g * L, L)]
                dr = plsc.load_gather(dinv_v, [r16])
                dc = plsc.load_gather(dinv_v, [c16])
                wch_v[pl.ds(g * L, L)] = dr * dc * weg_v[eb, jj, pl.ds(g * L, L)]

            def scale_row(e, _):
                e16 = jnp.full((L,), e, jnp.int32)
                ws = plsc.load_gather(wch_v, [e16])
                even = lax.iota(jnp.int32, L) * 2
                odd = even + 1
                for g in range(D // 32):
                    pk = xbuf_v[jj % 4, e, pl.ds(g * L, L)]
                    bf = plsc.bitcast(pk, jnp.bfloat16)
                    lo, hi = plsc.unpack(bf, format=plsc.PackFormat.INTERLEAVED)
                    plsc.store_scatter(sc_v, [e16, g * 32 + even], lo * ws)
                    plsc.store_scatter(sc_v, [e16, g * 32 + odd], hi * ws)
                return 0
            lax.fori_loop(0, CH, scale_row, 0)
            pltpu.sync_copy(sc_v, agg_sh.at[colg_v.at[eb, jj]], add=True)

        def main_group(g, eb, ebn):
            @pl.when(g + 2 < NG)
            def _():
                start_edge_group(g + 2, (eb + 2) % 3)

            @pl.when(g + 1 < NG)
            def _():
                wait_edge_group(g + 1, ebn)
            for jj in range(G):
                wait_gather(eb, jj)
                process(eb, jj)
                if jj < 4:
                    start_gather(eb, jj + 4)
                else:
                    @pl.when(g + 1 < NG)
                    def _():
                        start_gather(ebn, jj - 4)

        def main_tri(i, _):
            main_group(i * 3, 2, 0)
            main_group(i * 3 + 1, 0, 1)
            main_group(i * 3 + 2, 1, 2)
            return 0
        lax.fori_loop(0, (NG - 2) // 3, main_tri, 0)
        main_group(NG - 2, 2, 0)
        main_group(NG - 1, 0, 1)

        plsc.subcore_barrier()
        pltpu.sync_copy(agg_sh.at[pl.ds(sid * NPT, NPT)],
                        out_hbm.at[cid, pl.ds(sid * NPT, NPT)])

    return body(xw, colp, wp, rowp)


def _tc_body(x_ref, p0_ref, p1_ref, w_ref, b_ref, o_ref):
    h = x_ref[...] + p0_ref[0] + p1_ref[0]
    o_ref[...] = lax.dot_general(
        h, w_ref[...], (((1,), (1,)), ((), ())),
        preferred_element_type=jnp.float32) + b_ref[...]


def _tc_final(xp, partials, Wp, b2):
    blk = 400
    grid = (N // blk,)
    return pl.pallas_call(
        _tc_body,
        grid=grid,
        in_specs=[
            pl.BlockSpec((blk, D), lambda i: (i, 0)),
            pl.BlockSpec((1, blk, D), lambda i: (0, i, 0)),
            pl.BlockSpec((1, blk, D), lambda i: (1, i, 0)),
            pl.BlockSpec((D, D), lambda i: (0, 0)),
            pl.BlockSpec((1, D), lambda i: (0, 0)),
        ],
        out_specs=pl.BlockSpec((blk, D), lambda i: (i, 0)),
        out_shape=jax.ShapeDtypeStruct((N, D), jnp.float32),
    )(xp, partials, partials, Wp, b2)


def kernel(x, edge_index, weight, W, b):
    row = edge_index[0]
    col = edge_index[1]
    pad = EPAD - E
    rowp = jnp.pad(row, (0, pad)).reshape(NCHUNK, CH)
    colp = jnp.pad(col, (0, pad)).reshape(NCHUNK, CH)
    wp = jnp.pad(weight, (0, pad)).reshape(NCHUNK, CH)
    xw = lax.bitcast_convert_type(
        x.astype(jnp.bfloat16).reshape(N, DW, 2), jnp.int32)

    partials = _sc_aggregate(xw, colp, wp, rowp)
    return _tc_final(x, partials, W, b.reshape(1, D))


# speedup vs baseline: 14.2621x; 1.0580x over previous
import functools

import jax
import jax.numpy as jnp
from jax import lax
from jax.experimental import pallas as pl
from jax.experimental.pallas import tpu as pltpu, tpu_sc as plsc

N = 10000
E = 320000
D = 128

NC, NS, L = 2, 16, 16
NW = NC * NS
CH = 64
CPW = 160
EPAD = NW * CPW * CH
NCHUNK = NW * CPW
CPT_DEG = NCHUNK // NS
NPAD = 10240
NPT = NPAD // NS
G = 8
NG_DEG = CPT_DEG // G
NG = CPW // G
DW = D // 2


def _rsqrt_vec(d):
    i = plsc.bitcast(d, jnp.int32)
    i = jnp.full((L,), 0x5F3759DF, dtype=jnp.int32) - lax.shift_right_logical(i, 1)
    y = plsc.bitcast(i, jnp.float32)
    half_d = d * 0.5
    for _ in range(3):
        y = y * (1.5 - half_d * y * y)
    return jnp.where(d > 0.0, y, 0.0)


def _sc_aggregate(xw, eip, wp):
    mesh = plsc.VectorSubcoreMesh(core_axis_name="c", subcore_axis_name="s")

    @functools.partial(
        pl.kernel,
        mesh=mesh,
        out_type=jax.ShapeDtypeStruct((NC, NPAD, D), jnp.float32),
        scratch_types=[
            pltpu.VMEM((NPAD,), jnp.float32),
            pltpu.VMEM((CH,), jnp.float32),
            pltpu.VMEM((3, G, CH), jnp.int32),
            pltpu.VMEM((3, G, CH), jnp.int32),
            pltpu.VMEM((3, G, CH), jnp.float32),
            pltpu.VMEM((4, CH, DW), jnp.int32),
            pltpu.VMEM((CH, D), jnp.float32),
            pltpu.VMEM_SHARED((NPAD, D), jnp.float32),
            pltpu.VMEM_SHARED((NPAD,), jnp.float32),
            pltpu.VMEM_SHARED((NPAD,), jnp.float32),
            pltpu.SemaphoreType.DMA,
            pltpu.SemaphoreType.DMA,
            pltpu.SemaphoreType.DMA,
            pltpu.SemaphoreType.DMA,
            pltpu.SemaphoreType.DMA,
        ],
        compiler_params=pltpu.CompilerParams(
            needs_layout_passes=False, use_tc_tiling_on_sc=False),
    )
    def body(x_hbm, ei_hbm, w_hbm, out_hbm,
             dinv_v, wch_v, rowg_v, colg_v, weg_v, xbuf_v, sc_v,
             agg_sh, deg_sh, dinv_sh, semg0, semg1, semg2, semg3, seme):
        cid = lax.axis_index("c")
        sid = lax.axis_index("s")
        wid = sid * NC + cid
        gsems = [semg0, semg1, semg2, semg3]

        def start_edge_group(g, buf):
            base = wid * CPW + g * G
            pltpu.make_async_copy(
                ei_hbm.at[0, pl.ds(base, G)], rowg_v.at[buf], seme).start()
            pltpu.make_async_copy(
                ei_hbm.at[1, pl.ds(base, G)], colg_v.at[buf], seme).start()
            pltpu.make_async_copy(
                w_hbm.at[pl.ds(base, G)], weg_v.at[buf], seme).start()

        def wait_edge_group(g, buf):
            base = wid * CPW + g * G
            pltpu.make_async_copy(
                ei_hbm.at[0, pl.ds(base, G)], rowg_v.at[buf], seme).wait()
            pltpu.make_async_copy(
                ei_hbm.at[1, pl.ds(base, G)], colg_v.at[buf], seme).wait()
            pltpu.make_async_copy(
                w_hbm.at[pl.ds(base, G)], weg_v.at[buf], seme).wait()

        def start_gather(eb, jj):
            pltpu.make_async_copy(
                x_hbm.at[rowg_v.at[eb, jj]], xbuf_v.at[jj % 4],
                gsems[jj % 4]).start()

        def wait_gather(eb, jj):
            pltpu.make_async_copy(
                x_hbm.at[rowg_v.at[eb, jj]], xbuf_v.at[jj % 4],
                gsems[jj % 4]).wait()

        start_edge_group(0, 2)

        def zero_sc(r, _):
            for g in range(D // L):
                sc_v[r, pl.ds(g * L, L)] = jnp.zeros((L,), jnp.float32)
            return 0
        lax.fori_loop(0, CH, zero_sc, 0)
        for k in range(NPT // CH):
            pltpu.sync_copy(sc_v, agg_sh.at[pl.ds(sid * NPT + k * CH, CH)])
        for g in range(CH // L):
            wch_v[pl.ds(g * L, L)] = jnp.zeros((L,), jnp.float32)
        for k in range(NPT // CH):
            pltpu.sync_copy(wch_v, deg_sh.at[pl.ds(sid * NPT + k * CH, CH)])
        plsc.subcore_barrier()

        def start_deg_group(g, buf):
            base = sid * CPT_DEG + g * G
            pltpu.make_async_copy(
                ei_hbm.at[1, pl.ds(base, G)], colg_v.at[buf], seme).start()
            pltpu.make_async_copy(
                w_hbm.at[pl.ds(base, G)], weg_v.at[buf], seme).start()

        def wait_deg_group(g, buf):
            base = sid * CPT_DEG + g * G
            pltpu.make_async_copy(
                ei_hbm.at[1, pl.ds(base, G)], colg_v.at[buf], seme).wait()
            pltpu.make_async_copy(
                w_hbm.at[pl.ds(base, G)], weg_v.at[buf], seme).wait()

        def deg_group(g, buf):
            wait_deg_group(g, buf)
            for jj in range(G):
                pltpu.sync_copy(weg_v.at[buf, jj],
                                deg_sh.at[colg_v.at[buf, jj]], add=True)

            @pl.when(g + 2 < NG_DEG)
            def _():
                start_deg_group(g + 2, buf)

        start_deg_group(0, 0)
        start_deg_group(1, 1)
        wait_edge_group(0, 2)
        for jj in range(4):
            start_gather(2, jj)

        def deg_pair(i, _):
            deg_group(i * 2, 0)
            deg_group(i * 2 + 1, 1)
            return 0
        lax.fori_loop(0, NG_DEG // 2, deg_pair, 0)
        start_edge_group(1, 0)
        plsc.subcore_barrier()

        def dinv_piece(k, _):
            pltpu.sync_copy(deg_sh.at[pl.ds(sid * NPT + k * CH, CH)], wch_v)
            for g in range(CH // L):
                wch_v[pl.ds(g * L, L)] = _rsqrt_vec(wch_v[pl.ds(g * L, L)])
            pltpu.sync_copy(wch_v, dinv_sh.at[pl.ds(sid * NPT + k * CH, CH)])
            return 0
        lax.fori_loop(0, NPT // CH, dinv_piece, 0)
        plsc.subcore_barrier()
        pltpu.sync_copy(dinv_sh, dinv_v)

        def process(eb, jj):
            for g in range(CH // L):
                r16 = rowg_v[eb, jj, pl.ds(g * L, L)]
                c16 = colg_v[eb, jj, pl.ds(g * L, L)]
                dr = plsc.load_gather(dinv_v, [r16])
                dc = plsc.load_gather(dinv_v, [c16])
                wch_v[pl.ds(g * L, L)] = dr * dc * weg_v[eb, jj, pl.ds(g * L, L)]

            def scale_row(e, _):
                e16 = jnp.full((L,), e, jnp.int32)
                ws = plsc.load_gather(wch_v, [e16])
                even = lax.iota(jnp.int32, L) * 2
                odd = even + 1
                for g in range(D // 32):
                    pk = xbuf_v[jj % 4, e, pl.ds(g * L, L)]
                    bf = plsc.bitcast(pk, jnp.bfloat16)
                    lo, hi = plsc.unpack(bf, format=plsc.PackFormat.INTERLEAVED)
                    plsc.store_scatter(sc_v, [e16, g * 32 + even], lo * ws)
                    plsc.store_scatter(sc_v, [e16, g * 32 + odd], hi * ws)
                return 0
            lax.fori_loop(0, CH, scale_row, 0)
            pltpu.sync_copy(sc_v, agg_sh.at[colg_v.at[eb, jj]], add=True)

        def main_group(g, eb, ebn):
            @pl.when(g + 2 < NG)
            def _():
                start_edge_group(g + 2, (eb + 2) % 3)

            @pl.when(g + 1 < NG)
            def _():
                wait_edge_group(g + 1, ebn)
            for jj in range(G):
                wait_gather(eb, jj)
                process(eb, jj)
                if jj < 4:
                    start_gather(eb, jj + 4)
                else:
                    @pl.when(g + 1 < NG)
                    def _():
                        start_gather(ebn, jj - 4)

        def main_tri(i, _):
            main_group(i * 3, 2, 0)
            main_group(i * 3 + 1, 0, 1)
            main_group(i * 3 + 2, 1, 2)
            return 0
        lax.fori_loop(0, (NG - 2) // 3, main_tri, 0)
        main_group(NG - 2, 2, 0)
        main_group(NG - 1, 0, 1)

        plsc.subcore_barrier()
        pltpu.sync_copy(agg_sh.at[pl.ds(sid * NPT, NPT)],
                        out_hbm.at[cid, pl.ds(sid * NPT, NPT)])

    return body(xw, eip, wp)


def _tc_body(x_ref, p0_ref, p1_ref, w_ref, b_ref, o_ref):
    h = x_ref[...] + p0_ref[0] + p1_ref[0]
    o_ref[...] = lax.dot_general(
        h, w_ref[...], (((1,), (1,)), ((), ())),
        preferred_element_type=jnp.float32) + b_ref[...]


def _tc_final(xp, partials, Wp, b2):
    return pl.pallas_call(
        _tc_body,
        grid=(1,),
        in_specs=[
            pl.BlockSpec((N, D), lambda i: (0, 0)),
            pl.BlockSpec((1, N, D), lambda i: (0, 0, 0)),
            pl.BlockSpec((1, N, D), lambda i: (1, 0, 0)),
            pl.BlockSpec((D, D), lambda i: (0, 0)),
            pl.BlockSpec((1, D), lambda i: (0, 0)),
        ],
        out_specs=pl.BlockSpec((N, D), lambda i: (0, 0)),
        out_shape=jax.ShapeDtypeStruct((N, D), jnp.float32),
    )(xp, partials, partials, Wp, b2)


def kernel(x, edge_index, weight, W, b):
    pad = EPAD - E
    eip = jnp.pad(edge_index, ((0, 0), (0, pad))).reshape(2, NCHUNK, CH)
    wp = jnp.pad(weight, (0, pad)).reshape(NCHUNK, CH)
    xw = lax.bitcast_convert_type(
        x.astype(jnp.bfloat16).reshape(N, DW, 2), jnp.int32)

    partials = _sc_aggregate(xw, eip, wp)
    return _tc_final(x, partials, W, b.reshape(1, D))
